# Initial kernel scaffold; baseline (speedup 1.0000x reference)
import functools

import jax
import jax.numpy as jnp
from jax import lax
from jax.experimental import pallas as pl
from jax.experimental.pallas import tpu as pltpu
from jax.experimental.pallas import tpu_sc as plsc

N = 10000
E = 320000
R = 8
NB = 4

NUM_CORES = 2
NUM_SUBCORES = 16
NUM_WORKERS = NUM_CORES * NUM_SUBCORES
CHUNK = 128
CPW = -(-E // (NUM_WORKERS * CHUNK))
E_PAD = NUM_WORKERS * CPW * CHUNK

BN = 1000
N_BLOCKS = N // BN


def _rowidx_body(src_ref, et_ref, out_ref):
    out_ref[...] = src_ref[...] * R + et_ref[...]


def _compute_rowidx(src_p, etypes_p):
    rows = E_PAD // CHUNK
    f = pl.pallas_call(
        _rowidx_body,
        out_shape=jax.ShapeDtypeStruct((rows, CHUNK), jnp.int32),
    )
    return f(src_p.reshape(rows, CHUNK), etypes_p.reshape(rows, CHUNK)).reshape(E_PAD)


def _dense_first_body(h_ref, wb_ref, comp_ref, wl_ref, b_ref, hw_ref, loop_ref):
    h = h_ref[...]
    for r in range(R):
        w = comp_ref[r, 0] * wb_ref[0]
        for b in range(1, NB):
            w = w + comp_ref[r, b] * wb_ref[b]
        d = w.shape[1]
        hw_ref[:, r * d:(r + 1) * d] = jnp.dot(h, w, preferred_element_type=jnp.float32)
    loop_ref[...] = jnp.dot(h, wl_ref[...], preferred_element_type=jnp.float32) + b_ref[...]


def _dense_next_body(agg_ref, lp_ref, wb_ref, comp_ref, wl_ref, b_ref, hw_ref, loop_ref):
    x = agg_ref[0] + agg_ref[1] + lp_ref[...]
    h = jnp.where(x > 0, x, 0.01 * x)
    for r in range(R):
        w = comp_ref[r, 0] * wb_ref[0]
        for b in range(1, NB):
            w = w + comp_ref[r, b] * wb_ref[b]
        d = w.shape[1]
        hw_ref[:, r * d:(r + 1) * d] = jnp.dot(h, w, preferred_element_type=jnp.float32)
    loop_ref[...] = jnp.dot(h, wl_ref[...], preferred_element_type=jnp.float32) + b_ref[...]


def _dense_first(h, wb, comp, wl, b):
    din, d = wb.shape[1], wb.shape[2]
    f = pl.pallas_call(
        _dense_first_body,
        grid=(N_BLOCKS,),
        in_specs=[
            pl.BlockSpec((BN, din), lambda i: (i, 0)),
            pl.BlockSpec((NB, din, d), lambda i: (0, 0, 0)),
            pl.BlockSpec(memory_space=pltpu.SMEM),
            pl.BlockSpec((din, d), lambda i: (0, 0)),
            pl.BlockSpec((1, d), lambda i: (0, 0)),
        ],
        out_specs=[
            pl.BlockSpec((BN, R * d), lambda i: (i, 0)),
            pl.BlockSpec((BN, d), lambda i: (i, 0)),
        ],
        out_shape=[
            jax.ShapeDtypeStruct((N, R * d), jnp.float32),
            jax.ShapeDtypeStruct((N, d), jnp.float32),
        ],
    )
    return f(h, wb, comp, wl, b.reshape(1, d))


def _dense_next(agg, lp, wb, comp, wl, b):
    din, d = wb.shape[1], wb.shape[2]
    f = pl.pallas_call(
        _dense_next_body,
        grid=(N_BLOCKS,),
        in_specs=[
            pl.BlockSpec((2, BN, din), lambda i: (0, i, 0)),
            pl.BlockSpec((BN, din), lambda i: (i, 0)),
            pl.BlockSpec((NB, din, d), lambda i: (0, 0, 0)),
            pl.BlockSpec(memory_space=pltpu.SMEM),
            pl.BlockSpec((din, d), lambda i: (0, 0)),
            pl.BlockSpec((1, d), lambda i: (0, 0)),
        ],
        out_specs=[
            pl.BlockSpec((BN, R * d), lambda i: (i, 0)),
            pl.BlockSpec((BN, d), lambda i: (i, 0)),
        ],
        out_shape=[
            jax.ShapeDtypeStruct((N, R * d), jnp.float32),
            jax.ShapeDtypeStruct((N, d), jnp.float32),
        ],
    )
    return f(agg, lp, wb, comp, wl, b.reshape(1, d))


def _make_sc_scatter(d):
    mesh = plsc.VectorSubcoreMesh(core_axis_name="c", subcore_axis_name="s")
    rows_per_tile = N // NUM_SUBCORES

    def body(hw_hbm, idx_hbm, dst_hbm, norm_hbm, zeros_hbm, out_hbm,
             idxv, dstv, normv, rows, agg_sh, sem):
        c = lax.axis_index("c")
        s = lax.axis_index("s")
        wid = c * NUM_SUBCORES + s
        pltpu.sync_copy(zeros_hbm.at[pl.ds(s * rows_per_tile, rows_per_tile)],
                        agg_sh.at[pl.ds(s * rows_per_tile, rows_per_tile)])
        plsc.subcore_barrier()

        base_w = wid * (CPW * CHUNK)

        def chunk_body(ci, carry):
            base = base_w + ci * CHUNK
            pltpu.sync_copy(idx_hbm.at[pl.ds(base, CHUNK)], idxv)
            pltpu.sync_copy(dst_hbm.at[pl.ds(base, CHUNK)], dstv)
            pltpu.sync_copy(norm_hbm.at[pl.ds(base, CHUNK)], normv)
            pltpu.async_copy(hw_hbm.at[idxv], rows, sem).wait()
            for g in range(CHUNK // 16):
                nv = normv[pl.ds(g * 16, 16)]
                for l in range(16):
                    e = g * 16 + l
                    sc = jnp.broadcast_to(nv[l], (16,))
                    for j in range(d // 16):
                        rows[e, pl.ds(j * 16, 16)] = rows[e, pl.ds(j * 16, 16)] * sc
            pltpu.sync_copy(rows, agg_sh.at[dstv], add=True)
            return carry

        lax.fori_loop(0, CPW, chunk_body, 0)
        plsc.subcore_barrier()
        pltpu.sync_copy(agg_sh.at[pl.ds(s * rows_per_tile, rows_per_tile)],
                        out_hbm.at[c, pl.ds(s * rows_per_tile, rows_per_tile)])

    return pl.kernel(
        body,
        out_type=jax.ShapeDtypeStruct((NUM_CORES, N, d), jnp.float32),
        mesh=mesh,
        scratch_types=[
            pltpu.VMEM((CHUNK,), jnp.int32),
            pltpu.VMEM((CHUNK,), jnp.int32),
            pltpu.VMEM((CHUNK,), jnp.float32),
            pltpu.VMEM((CHUNK, d), jnp.float32),
            pltpu.VMEM_SHARED((N, d), jnp.float32),
            pltpu.SemaphoreType.DMA,
        ],
    )


_sc_scatter_cache = {}


def _sc_scatter(d, hw_flat, rowidx, dst_p, norm_p):
    if d not in _sc_scatter_cache:
        _sc_scatter_cache[d] = _make_sc_scatter(d)
    zeros = jnp.zeros((N, d), jnp.float32)
    return _sc_scatter_cache[d](hw_flat, rowidx, dst_p, norm_p, zeros)


def _readout_body(a_ref, l_ref, o_ref):
    o_ref[...] = jnp.tanh(a_ref[0] + a_ref[1] + l_ref[...])


def _readout(a, l):
    f = pl.pallas_call(
        _readout_body,
        out_shape=jax.ShapeDtypeStruct(l.shape, jnp.float32),
    )
    return f(a, l)


def kernel(features, etypes, edge_index, norm,
           Wb1, comp1, Wloop1, b1,
           Wb2, comp2, Wloop2, b2,
           Wb3, comp3, Wloop3, b3):
    src = edge_index[0].astype(jnp.int32)
    dst = edge_index[1].astype(jnp.int32)
    et = etypes.astype(jnp.int32)
    pad = E_PAD - E
    src_p = jnp.pad(src, (0, pad))
    dst_p = jnp.pad(dst, (0, pad))
    et_p = jnp.pad(et, (0, pad))
    norm_p = jnp.pad(norm.reshape(E), (0, pad))

    rowidx = _compute_rowidx(src_p, et_p)

    d3 = 16
    Wb3p = jnp.pad(Wb3, ((0, 0), (0, 0), (0, d3 - Wb3.shape[2])))
    Wloop3p = jnp.pad(Wloop3, ((0, 0), (0, d3 - Wloop3.shape[1])))
    b3p = jnp.pad(b3, (0, d3 - b3.shape[0]))

    hw1, loop1 = _dense_first(features, Wb1, comp1, Wloop1, b1)
    agg1 = _sc_scatter(128, hw1.reshape(N * R, 128), rowidx, dst_p, norm_p)

    hw2, loop2 = _dense_next(agg1, loop1, Wb2, comp2, Wloop2, b2)
    agg2 = _sc_scatter(64, hw2.reshape(N * R, 64), rowidx, dst_p, norm_p)

    hw3, loop3 = _dense_next(agg2, loop2, Wb3p, comp3, Wloop3p, b3p)
    agg3 = _sc_scatter(d3, hw3.reshape(N * R, d3), rowidx, dst_p, norm_p)

    offsets = jnp.arange(16) * (N // 16)
    a = agg3[:, offsets, :]
    l = loop3[offsets, :]
    out = _readout(a, l)
    return out[:, :3]


# speedup vs baseline: 15.6272x; 15.6272x over previous
import functools

import jax
import jax.numpy as jnp
from jax import lax
from jax.experimental import pallas as pl
from jax.experimental.pallas import tpu as pltpu
from jax.experimental.pallas import tpu_sc as plsc

N = 10000
E = 320000
R = 8
NB = 4

NUM_CORES = 2
NUM_SUBCORES = 16
NUM_WORKERS = NUM_CORES * NUM_SUBCORES
CHUNK = 128
CPW = -(-E // (NUM_WORKERS * CHUNK))
E_PAD = NUM_WORKERS * CPW * CHUNK

BN = 1000
N_BLOCKS = N // BN

N_PAD = 10240


def _rowidx_body(src_ref, et_ref, out_ref):
    out_ref[...] = src_ref[...] * R + et_ref[...]


def _compute_rowidx(src_p, etypes_p):
    rows = E_PAD // CHUNK
    f = pl.pallas_call(
        _rowidx_body,
        out_shape=jax.ShapeDtypeStruct((rows, CHUNK), jnp.int32),
    )
    return f(src_p.reshape(rows, CHUNK), etypes_p.reshape(rows, CHUNK)).reshape(E_PAD)


def _dense_first_body(h_ref, wb_ref, comp_ref, wl_ref, b_ref, hw_ref, loop_ref):
    h = h_ref[...]
    for r in range(R):
        w = comp_ref[r, 0] * wb_ref[0]
        for b in range(1, NB):
            w = w + comp_ref[r, b] * wb_ref[b]
        d = w.shape[1]
        hw_ref[:, r * d:(r + 1) * d] = jnp.dot(h, w, preferred_element_type=jnp.float32)
    loop_ref[...] = jnp.dot(h, wl_ref[...], preferred_element_type=jnp.float32) + b_ref[...]


def _dense_next_body(agg_ref, lp_ref, wb_ref, comp_ref, wl_ref, b_ref, hw_ref, loop_ref):
    x = agg_ref[0] + agg_ref[1] + lp_ref[...]
    h = jnp.where(x > 0, x, 0.01 * x)
    for r in range(R):
        w = comp_ref[r, 0] * wb_ref[0]
        for b in range(1, NB):
            w = w + comp_ref[r, b] * wb_ref[b]
        d = w.shape[1]
        hw_ref[:, r * d:(r + 1) * d] = jnp.dot(h, w, preferred_element_type=jnp.float32)
    loop_ref[...] = jnp.dot(h, wl_ref[...], preferred_element_type=jnp.float32) + b_ref[...]


def _dense_first(h, wb, comp, wl, b):
    din, d = wb.shape[1], wb.shape[2]
    f = pl.pallas_call(
        _dense_first_body,
        grid=(N_BLOCKS,),
        in_specs=[
            pl.BlockSpec((BN, din), lambda i: (i, 0)),
            pl.BlockSpec((NB, din, d), lambda i: (0, 0, 0)),
            pl.BlockSpec(memory_space=pltpu.SMEM),
            pl.BlockSpec((din, d), lambda i: (0, 0)),
            pl.BlockSpec((1, d), lambda i: (0, 0)),
        ],
        out_specs=[
            pl.BlockSpec((BN, R * d), lambda i: (i, 0)),
            pl.BlockSpec((BN, d), lambda i: (i, 0)),
        ],
        out_shape=[
            jax.ShapeDtypeStruct((N, R * d), jnp.float32),
            jax.ShapeDtypeStruct((N, d), jnp.float32),
        ],
    )
    return f(h, wb, comp, wl, b.reshape(1, d))


def _dense_next(agg, lp, wb, comp, wl, b):
    din, d = wb.shape[1], wb.shape[2]
    f = pl.pallas_call(
        _dense_next_body,
        grid=(N_BLOCKS,),
        in_specs=[
            pl.BlockSpec((2, BN, din), lambda i: (0, i, 0)),
            pl.BlockSpec((BN, din), lambda i: (i, 0)),
            pl.BlockSpec((NB, din, d), lambda i: (0, 0, 0)),
            pl.BlockSpec(memory_space=pltpu.SMEM),
            pl.BlockSpec((din, d), lambda i: (0, 0)),
            pl.BlockSpec((1, d), lambda i: (0, 0)),
        ],
        out_specs=[
            pl.BlockSpec((BN, R * d), lambda i: (i, 0)),
            pl.BlockSpec((BN, d), lambda i: (i, 0)),
        ],
        out_shape=[
            jax.ShapeDtypeStruct((N, R * d), jnp.float32),
            jax.ShapeDtypeStruct((N, d), jnp.float32),
        ],
    )
    return f(agg, lp, wb, comp, wl, b.reshape(1, d))


def _make_sc_scatter(d):
    mesh = plsc.VectorSubcoreMesh(core_axis_name="c", subcore_axis_name="s")
    rows_per_tile = N_PAD // NUM_SUBCORES

    def body(hw_hbm, idx_hbm, dst_hbm, norm_hbm, zeros_hbm, out_hbm,
             idxv, dstv, normv, rows, agg_sh, sem):
        c = lax.axis_index("c")
        s = lax.axis_index("s")
        wid = c * NUM_SUBCORES + s
        pltpu.sync_copy(zeros_hbm.at[pl.ds(s * rows_per_tile, rows_per_tile)],
                        agg_sh.at[pl.ds(s * rows_per_tile, rows_per_tile)])
        plsc.subcore_barrier()

        base_w = wid * (CPW * CHUNK)

        def chunk_body(ci, carry):
            base = base_w + ci * CHUNK
            pltpu.sync_copy(idx_hbm.at[pl.ds(base, CHUNK)], idxv)
            pltpu.sync_copy(dst_hbm.at[pl.ds(base, CHUNK)], dstv)
            pltpu.sync_copy(norm_hbm.at[pl.ds(base, CHUNK)], normv)
            pltpu.async_copy(hw_hbm.at[idxv], rows, sem).wait()
            for g in range(CHUNK // 16):
                nv = normv[pl.ds(g * 16, 16)]
                for l in range(16):
                    e = g * 16 + l
                    sc = jnp.broadcast_to(nv[l], (16,))
                    for j in range(d // 16):
                        rows[e, pl.ds(j * 16, 16)] = rows[e, pl.ds(j * 16, 16)] * sc
            pltpu.sync_copy(rows, agg_sh.at[dstv], add=True)
            return carry

        lax.fori_loop(0, CPW, chunk_body, 0)
        plsc.subcore_barrier()
        pltpu.sync_copy(agg_sh.at[pl.ds(s * rows_per_tile, rows_per_tile)],
                        out_hbm.at[c, pl.ds(s * rows_per_tile, rows_per_tile)])

    return pl.kernel(
        body,
        out_type=jax.ShapeDtypeStruct((NUM_CORES, N_PAD, d), jnp.float32),
        mesh=mesh,
        compiler_params=pltpu.CompilerParams(use_tc_tiling_on_sc=False),
        scratch_types=[
            pltpu.VMEM((CHUNK,), jnp.int32),
            pltpu.VMEM((CHUNK,), jnp.int32),
            pltpu.VMEM((CHUNK,), jnp.float32),
            pltpu.VMEM((CHUNK, d), jnp.float32),
            pltpu.VMEM_SHARED((N_PAD, d), jnp.float32),
            pltpu.SemaphoreType.DMA,
        ],
    )


_sc_scatter_cache = {}


def _sc_scatter(d, hw_flat, rowidx, dst_p, norm_p):
    if d not in _sc_scatter_cache:
        _sc_scatter_cache[d] = _make_sc_scatter(d)
    zeros = jnp.zeros((N_PAD, d), jnp.float32)
    return _sc_scatter_cache[d](hw_flat, rowidx, dst_p, norm_p, zeros)


def _readout_body(a_ref, l_ref, o_ref):
    o_ref[...] = jnp.tanh(a_ref[0] + a_ref[1] + l_ref[...])


def _readout(a, l):
    f = pl.pallas_call(
        _readout_body,
        out_shape=jax.ShapeDtypeStruct(l.shape, jnp.float32),
    )
    return f(a, l)


def kernel(features, etypes, edge_index, norm,
           Wb1, comp1, Wloop1, b1,
           Wb2, comp2, Wloop2, b2,
           Wb3, comp3, Wloop3, b3):
    src = edge_index[0].astype(jnp.int32)
    dst = edge_index[1].astype(jnp.int32)
    et = etypes.astype(jnp.int32)
    pad = E_PAD - E
    src_p = jnp.pad(src, (0, pad))
    dst_p = jnp.pad(dst, (0, pad))
    et_p = jnp.pad(et, (0, pad))
    norm_p = jnp.pad(norm.reshape(E), (0, pad))

    rowidx = _compute_rowidx(src_p, et_p)

    d3 = 16
    Wb3p = jnp.pad(Wb3, ((0, 0), (0, 0), (0, d3 - Wb3.shape[2])))
    Wloop3p = jnp.pad(Wloop3, ((0, 0), (0, d3 - Wloop3.shape[1])))
    b3p = jnp.pad(b3, (0, d3 - b3.shape[0]))

    hw1, loop1 = _dense_first(features, Wb1, comp1, Wloop1, b1)
    agg1 = _sc_scatter(128, hw1.reshape(N * R, 128), rowidx, dst_p, norm_p)

    hw2, loop2 = _dense_next(agg1, loop1, Wb2, comp2, Wloop2, b2)
    agg2 = _sc_scatter(64, hw2.reshape(N * R, 64), rowidx, dst_p, norm_p)

    hw3, loop3 = _dense_next(agg2, loop2, Wb3p, comp3, Wloop3p, b3p)
    agg3 = _sc_scatter(d3, hw3.reshape(N * R, d3), rowidx, dst_p, norm_p)

    offsets = jnp.arange(16) * (N // 16)
    a = agg3[:, offsets, :]
    l = loop3[offsets, :]
    out = _readout(a, l)
    return out[:, :3]
